# Initial kernel scaffold; baseline (speedup 1.0000x reference)
import jax
import jax.numpy as jnp
from jax.experimental import pallas as pl


def kernel(x, xl, U):
    raise NotImplementedError("write your pallas kernel here")


# speedup vs baseline: 68.0868x; 68.0868x over previous
import functools

import jax
import jax.numpy as jnp
from jax import lax
from jax.experimental import pallas as pl
from jax.experimental.pallas import tpu as pltpu
from jax.experimental.pallas import tpu_sc as plsc

NC = 2
NS = 16
NW = NC * NS

LANES = 16
GSLICE = 128

LN2 = 0.6931471805599453


def _make_sc_call(B, S, K, V):
    rows_per_w = B // NW
    ch = 32
    nchunk = rows_per_w // ch
    chunk_elems = ch * S
    ng = chunk_elems // GSLICE
    nvreg = (S + LANES - 1) // LANES
    tail = S - (nvreg - 1) * LANES
    pad = LANES

    mesh = plsc.VectorSubcoreMesh(core_axis_name="c", subcore_axis_name="s")

    @functools.partial(
        pl.kernel,
        out_type=jax.ShapeDtypeStruct((B * S,), jnp.float32),
        mesh=mesh,
        scratch_types=[
            pltpu.VMEM((rows_per_w + LANES,), jnp.int32),
            pltpu.VMEM((chunk_elems + pad,), jnp.int32),
            pltpu.VMEM((chunk_elems + pad,), jnp.int32),
            pltpu.VMEM((chunk_elems + pad,), jnp.float32),
            pltpu.VMEM((chunk_elems + pad,), jnp.float32),
            pltpu.SemaphoreType.DMA,
        ],
    )
    def sc_call(x_hbm, xl_hbm, u_hbm, out_hbm,
                xlbuf, xbuf, idxbuf, valbuf, outbuf, gsem):
        cid = lax.axis_index("c")
        sid = lax.axis_index("s")
        wid = sid * NC + cid
        row0 = wid * rows_per_w

        pltpu.sync_copy(xl_hbm.at[pl.ds(row0, rows_per_w)],
                        xlbuf.at[pl.ds(0, rows_per_w)])

        lanes = lax.iota(jnp.int32, LANES)
        tail_mask = lanes < tail

        def chunk_body(ci, _):
            base = (row0 + ci * ch) * S
            pltpu.sync_copy(x_hbm.at[pl.ds(base, chunk_elems)],
                            xbuf.at[pl.ds(0, chunk_elems)])

            def row_idx(r, _):
                xlv = jnp.clip(xlbuf[pl.ds(ci * ch + r, LANES)][0], 0, K - 1)
                off = r * S
                for j in range(nvreg):
                    xv = xbuf[pl.ds(off + LANES * j, LANES)]
                    idxbuf[pl.ds(off + LANES * j, LANES)] = xv * K + xlv
                return 0
            lax.fori_loop(0, ch, row_idx, 0, unroll=False)

            def fire(j, _):
                pltpu.async_copy(
                    u_hbm.at[idxbuf.at[pl.ds(j * GSLICE, GSLICE)]],
                    valbuf.at[pl.ds(j * GSLICE, GSLICE)],
                    gsem)
                return 0
            lax.fori_loop(0, ng, fire, 0, unroll=False)

            def drain(j, _):
                pltpu.make_async_copy(
                    u_hbm.at[idxbuf.at[pl.ds(j * GSLICE, GSLICE)]],
                    valbuf.at[pl.ds(j * GSLICE, GSLICE)],
                    gsem).wait()
                return 0
            lax.fori_loop(0, ng, drain, 0, unroll=False)

            def row_lse(r, _):
                off = r * S
                sv = jnp.zeros((LANES,), jnp.float32)
                for j in range(nvreg):
                    v = valbuf[pl.ds(off + LANES * j, LANES)]
                    e = jnp.exp(v)
                    if j == nvreg - 1:
                        e = jnp.where(tail_mask, e, 0.0)
                    sv = sv + e
                stot = sv[0]
                for i in range(1, LANES):
                    stot = stot + sv[i]
                sb = jnp.full((LANES,), stot)
                bits = lax.bitcast_convert_type(sb, jnp.int32)
                y = (bits.astype(jnp.float32) * jnp.float32(1.1920929e-7)
                     - 127.0) * jnp.float32(LN2)
                y = y - 1.0 + sb * jnp.exp(-y)
                y = y - 1.0 + sb * jnp.exp(-y)
                for j in range(nvreg):
                    v = valbuf[pl.ds(off + LANES * j, LANES)]
                    outbuf[pl.ds(off + LANES * j, LANES)] = v - y
                return 0
            lax.fori_loop(0, ch, row_lse, 0, unroll=False)

            pltpu.sync_copy(outbuf.at[pl.ds(0, chunk_elems)],
                            out_hbm.at[pl.ds(base, chunk_elems)])
            return 0

        lax.fori_loop(0, nchunk, chunk_body, 0, unroll=False)

    return sc_call


def kernel(x, xl, U):
    B, S = x.shape
    V, K = U.shape
    sc_call = _make_sc_call(B, S, K, V)
    out = sc_call(x.reshape(-1).astype(jnp.int32),
                  xl.astype(jnp.int32),
                  U.reshape(-1))
    return out.reshape(B, S)


# speedup vs baseline: 77.3427x; 1.1359x over previous
import functools

import jax
import jax.numpy as jnp
from jax import lax
from jax.experimental import pallas as pl
from jax.experimental.pallas import tpu as pltpu
from jax.experimental.pallas import tpu_sc as plsc

NC = 2
NS = 16
NW = NC * NS

LANES = 16
GSLICE = 128

LN2 = 0.6931471805599453


def _make_sc_call(B, S, K, V):
    rows_per_w = B // NW
    ch = 32
    nchunk = rows_per_w // ch
    chunk_elems = ch * S
    ng = chunk_elems // GSLICE
    nvreg = (S + LANES - 1) // LANES
    tail = S - (nvreg - 1) * LANES
    pad = LANES

    mesh = plsc.VectorSubcoreMesh(core_axis_name="c", subcore_axis_name="s")

    @functools.partial(
        pl.kernel,
        out_type=jax.ShapeDtypeStruct((B * S,), jnp.float32),
        mesh=mesh,
        scratch_types=[
            pltpu.VMEM((rows_per_w + LANES,), jnp.int32),
            pltpu.VMEM((2 * (chunk_elems + pad),), jnp.int32),
            pltpu.VMEM((2 * (chunk_elems + pad),), jnp.int32),
            pltpu.VMEM((2 * (chunk_elems + pad),), jnp.float32),
            pltpu.VMEM((2 * (chunk_elems + pad),), jnp.float32),
            pltpu.SemaphoreType.DMA,
            pltpu.SemaphoreType.DMA,
            pltpu.SemaphoreType.DMA,
        ],
    )
    def sc_call(x_hbm, xl_hbm, u_hbm, out_hbm,
                xlbuf, xbuf, idxbuf, valbuf, outbuf, gsem, xsem, osem):
        cid = lax.axis_index("c")
        sid = lax.axis_index("s")
        wid = sid * NC + cid
        row0 = wid * rows_per_w

        pltpu.sync_copy(xl_hbm.at[pl.ds(row0, rows_per_w)],
                        xlbuf.at[pl.ds(0, rows_per_w)])

        lanes = lax.iota(jnp.int32, LANES)
        tail_mask = lanes < tail
        stride = chunk_elems + pad

        def chunk_base(ci):
            return (row0 + ci * ch) * S

        def pass1(ci, p):
            def row_idx(r, _):
                xlv = jnp.clip(xlbuf[pl.ds(ci * ch + r, LANES)][0], 0, K - 1)
                off = p * stride + r * S
                for j in range(nvreg):
                    xv = xbuf[pl.ds(off + LANES * j, LANES)]
                    idxbuf[pl.ds(off + LANES * j, LANES)] = xv * K + xlv
                return 0
            lax.fori_loop(0, ch, row_idx, 0)

        def fire(p):
            def body(j, _):
                pltpu.async_copy(
                    u_hbm.at[idxbuf.at[pl.ds(p * stride + j * GSLICE, GSLICE)]],
                    valbuf.at[pl.ds(p * stride + j * GSLICE, GSLICE)],
                    gsem)
                return 0
            lax.fori_loop(0, ng, body, 0)

        def drain(p):
            pltpu.make_async_copy(
                u_hbm.at[pl.ds(0, chunk_elems)],
                valbuf.at[pl.ds(p * stride, chunk_elems)],
                gsem).wait()

        def pass2(ci, p):
            def row_lse(r, _):
                off = p * stride + r * S
                sv = jnp.zeros((LANES,), jnp.float32)
                for j in range(nvreg):
                    v = valbuf[pl.ds(off + LANES * j, LANES)]
                    e = jnp.exp(v)
                    if j == nvreg - 1:
                        e = jnp.where(tail_mask, e, 0.0)
                    sv = sv + e
                stot = sv[0]
                for i in range(1, LANES):
                    stot = stot + sv[i]
                sb = jnp.full((LANES,), stot)
                bits = lax.bitcast_convert_type(sb, jnp.int32)
                y = (bits.astype(jnp.float32) * jnp.float32(1.1920929e-7)
                     - 127.0) * jnp.float32(LN2)
                y = y - 1.0 + sb * jnp.exp(-y)
                y = y - 1.0 + sb * jnp.exp(-y)
                for j in range(nvreg):
                    v = valbuf[pl.ds(off + LANES * j, LANES)]
                    outbuf[pl.ds(off + LANES * j, LANES)] = v - y
                return 0
            lax.fori_loop(0, ch, row_lse, 0)

        def load_x(ci, p, sem):
            return pltpu.async_copy(
                x_hbm.at[pl.ds(chunk_base(ci), chunk_elems)],
                xbuf.at[pl.ds(p * stride, chunk_elems)],
                sem)

        pltpu.sync_copy(x_hbm.at[pl.ds(chunk_base(0), chunk_elems)],
                        xbuf.at[pl.ds(0, chunk_elems)])
        pass1(0, 0)
        fire(0)
        load_x(1, 1, xsem)

        def chunk_body(ci, _):
            p = lax.rem(ci, 2)
            q = 1 - p

            @pl.when(ci + 1 < nchunk)
            def _():
                pltpu.make_async_copy(
                    x_hbm.at[pl.ds(0, chunk_elems)],
                    xbuf.at[pl.ds(q * stride, chunk_elems)],
                    xsem).wait()
                pass1(ci + 1, q)

            drain(p)

            @pl.when(ci + 1 < nchunk)
            def _():
                fire(q)

            @pl.when(ci + 2 < nchunk)
            def _():
                load_x(ci + 2, p, xsem)

            @pl.when(ci >= 2)
            def _():
                pltpu.make_async_copy(
                    outbuf.at[pl.ds(p * stride, chunk_elems)],
                    out_hbm.at[pl.ds(0, chunk_elems)],
                    osem).wait()

            pass2(ci, p)
            pltpu.async_copy(
                outbuf.at[pl.ds(p * stride, chunk_elems)],
                out_hbm.at[pl.ds(chunk_base(ci), chunk_elems)],
                osem)
            return 0

        lax.fori_loop(0, nchunk, chunk_body, 0)

        for _ in range(2):
            pltpu.make_async_copy(
                outbuf.at[pl.ds(0, chunk_elems)],
                out_hbm.at[pl.ds(0, chunk_elems)],
                osem).wait()

    return sc_call


def kernel(x, xl, U):
    B, S = x.shape
    V, K = U.shape
    sc_call = _make_sc_call(B, S, K, V)
    out = sc_call(x.reshape(-1).astype(jnp.int32),
                  xl.astype(jnp.int32),
                  U.reshape(-1))
    return out.reshape(B, S)
